# baseline (speedup 1.0000x reference)
import functools
import math

import jax
import jax.numpy as jnp
from jax import lax
from jax.experimental import pallas as pl
from jax.experimental.pallas import tpu as pltpu
from jax.experimental.pallas import tpu_sc as plsc

NUM_EMB = 1000000
DIM = 64
SCALE = math.sqrt(DIM)

_info = plsc.get_sparse_core_info()
NC, NS, L = _info.num_cores, _info.num_subcores, _info.num_lanes
NW = NC * NS


def _make_kernel(B, C):
    b_per_w = B // NW
    n_chunks = b_per_w // C
    mesh = plsc.VectorSubcoreMesh(core_axis_name="c", subcore_axis_name="s")

    @functools.partial(
        pl.kernel,
        mesh=mesh,
        out_type=jax.ShapeDtypeStruct((B, DIM), jnp.float32),
        scratch_types=[
            pltpu.VMEM((C,), jnp.int32),
            pltpu.VMEM((C, DIM), jnp.float32),
            pltpu.SemaphoreType.DMA,
        ],
        compiler_params=pltpu.CompilerParams(use_tc_tiling_on_sc=False),
    )
    def k(x_hbm, table_hbm, out_hbm, idx_v, rows_v, sem):
        wid = lax.axis_index("s") * NC + lax.axis_index("c")
        base = wid * b_per_w

        def chunk_body(i, carry):
            off = base + i * C
            pltpu.sync_copy(x_hbm.at[pl.ds(off, C)], idx_v)
            pltpu.async_copy(table_hbm.at[idx_v], rows_v, sem).wait()

            def scale_row(r, c2):
                for c4 in range(DIM // L):
                    sl = pl.ds(c4 * L, L)
                    rows_v[r, sl] = rows_v[r, sl] * SCALE
                return c2

            lax.fori_loop(0, C, scale_row, 0)
            pltpu.sync_copy(rows_v, out_hbm.at[pl.ds(off, C)])
            return carry

        lax.fori_loop(0, n_chunks, chunk_body, 0)

    return k


@jax.jit
def kernel(x, table):
    B = x.shape[0] * x.shape[1]
    flat = jnp.asarray(x, jnp.int32).reshape(B)
    out = _make_kernel(B, 512)(flat, table)
    return out.reshape(x.shape[0], x.shape[1], DIM)


# speedup vs baseline: 1.1356x; 1.1356x over previous
import functools
import math

import jax
import jax.numpy as jnp
from jax import lax
from jax.experimental import pallas as pl
from jax.experimental.pallas import tpu as pltpu
from jax.experimental.pallas import tpu_sc as plsc

NUM_EMB = 1000000
DIM = 64
SCALE = math.sqrt(DIM)

_info = plsc.get_sparse_core_info()
NC, NS, L = _info.num_cores, _info.num_subcores, _info.num_lanes
NW = NC * NS

NBUF = 4
UNROLL = 8


def _make_kernel(B, C):
    b_per_w = B // NW
    n_chunks = b_per_w // C
    assert b_per_w % C == 0 and n_chunks % NBUF == 0 and C % UNROLL == 0
    n_outer = n_chunks // NBUF
    mesh = plsc.VectorSubcoreMesh(core_axis_name="c", subcore_axis_name="s")

    @functools.partial(
        pl.kernel,
        mesh=mesh,
        out_type=jax.ShapeDtypeStruct((B, DIM), jnp.float32),
        scratch_types=[
            pltpu.VMEM((b_per_w,), jnp.int32),
            pltpu.VMEM((NBUF, C, DIM), jnp.float32),
            pltpu.SemaphoreType.DMA((NBUF,)),
            pltpu.SemaphoreType.DMA((NBUF,)),
        ],
        compiler_params=pltpu.CompilerParams(use_tc_tiling_on_sc=False),
    )
    def k(x_hbm, table_hbm, out_hbm, idx_v, rows_v, gsem, ssem):
        wid = lax.axis_index("s") * NC + lax.axis_index("c")
        base = wid * b_per_w
        pltpu.sync_copy(x_hbm.at[pl.ds(base, b_per_w)], idx_v)

        def start_gather(g, b):
            pltpu.make_async_copy(
                table_hbm.at[idx_v.at[pl.ds(g * C, C)]],
                rows_v.at[b],
                gsem.at[b],
            ).start()

        start_gather(0, 0)
        start_gather(1, 1)

        def outer(o, carry):
            for j in range(NBUF):
                g = o * NBUF + j

                bn = (j + 2) % NBUF

                @pl.when(g >= 2)
                def _():
                    pltpu.make_async_copy(
                        rows_v.at[bn], out_hbm.at[pl.ds(0, C)], ssem.at[bn]
                    ).wait()

                @pl.when(g + 2 < n_chunks)
                def _():
                    start_gather(g + 2, bn)

                pltpu.make_async_copy(
                    table_hbm.at[idx_v.at[pl.ds(0, C)]],
                    rows_v.at[j],
                    gsem.at[j],
                ).wait()

                rows = rows_v.at[j]

                def scale(r0, c2):
                    for u in range(UNROLL):
                        for c4 in range(DIM // L):
                            sl = pl.ds(c4 * L, L)
                            rows[r0 * UNROLL + u, sl] = (
                                rows[r0 * UNROLL + u, sl] * SCALE
                            )
                    return c2

                lax.fori_loop(0, C // UNROLL, scale, 0, unroll=False)
                pltpu.make_async_copy(
                    rows_v.at[j],
                    out_hbm.at[pl.ds(base + g * C, C)],
                    ssem.at[j],
                ).start()
            return carry

        lax.fori_loop(0, n_outer, outer, 0)

        for j in (NBUF - 2, NBUF - 1):
            pltpu.make_async_copy(
                rows_v.at[j], out_hbm.at[pl.ds(0, C)], ssem.at[j]
            ).wait()

    return k


@jax.jit
def kernel(x, table):
    B = x.shape[0] * x.shape[1]
    flat = jnp.asarray(x, jnp.int32).reshape(B)
    out = _make_kernel(B, 400)(flat, table)
    return out.reshape(x.shape[0], x.shape[1], DIM)
